# baseline (speedup 1.0000x reference)
import functools

import jax
import jax.numpy as jnp
from jax import lax
from jax.experimental import pallas as pl
from jax.experimental.pallas import tpu as pltpu, tpu_sc as plsc

D_MODEL = 64
SEQ_LEN = 1024
NUM_CORES = 2
NUM_SUBCORES = 16
NUM_WORKERS = NUM_CORES * NUM_SUBCORES
LANES = 16
CHUNK = 128
CHUNKS_PER_SEQ = SEQ_LEN // CHUNK


def _body(tok_hbm, te_hbm, pe_hbm, out_hbm,
          toks_v, pos_v, bt, bp, carry_v,
          sem_t0, sem_t1, sem_p0, sem_p1, sem_o0, sem_o1, seq_per_worker):
    wid = lax.axis_index("s") * NUM_CORES + lax.axis_index("c")
    sem_t = (sem_t0, sem_t1)
    sem_p = (sem_p0, sem_p1)
    sem_o = (sem_o0, sem_o1)

    def per_seq(i, _):
        s = wid * seq_per_worker + i
        base_tok = s * SEQ_LEN
        pltpu.sync_copy(tok_hbm.at[pl.ds(base_tok, SEQ_LEN)], toks_v)

        carry_v[...] = jnp.zeros((LANES,), jnp.int32)

        def pos_grp(g, _):
            t16 = toks_v[pl.ds(g * LANES, LANES)]
            m = jnp.minimum(t16, 1)
            cs = plsc.cumsum(m)
            carry = carry_v[...]
            pos_v[pl.ds(g * LANES, LANES)] = (cs + carry) * m
            carry_v[...] = carry + lax.reduce_sum(m, axes=(0,))
            return 0

        lax.fori_loop(0, SEQ_LEN // LANES, pos_grp, 0)

        cp_t = [None, None]
        cp_p = [None, None]
        cp_o = [None, None]

        def issue(c):
            slot = c & 1
            base = c * CHUNK
            cp_t[slot] = pltpu.async_copy(
                te_hbm.at[toks_v.at[pl.ds(base, CHUNK)]],
                bt.at[slot], sem_t[slot])
            cp_p[slot] = pltpu.async_copy(
                pe_hbm.at[pos_v.at[pl.ds(base, CHUNK)]],
                bp.at[slot], sem_p[slot])

        issue(0)
        for c in range(CHUNKS_PER_SEQ):
            slot = c & 1
            other = 1 - slot
            if c + 1 < CHUNKS_PER_SEQ:
                if cp_o[other] is not None:
                    cp_o[other].wait()
                issue(c + 1)
            cp_t[slot].wait()
            cp_p[slot].wait()

            @plsc.parallel_loop(0, CHUNK, step=1, unroll=8)
            def add_row(r):
                for j in range(D_MODEL // LANES):
                    sl = pl.ds(j * LANES, LANES)
                    bt[slot, r, sl] = bt[slot, r, sl] + bp[slot, r, sl]

            cp_o[slot] = pltpu.async_copy(
                bt.at[slot], out_hbm.at[pl.ds(base_tok + c * CHUNK, CHUNK)],
                sem_o[slot])
        cp_o[0].wait()
        cp_o[1].wait()
        return 0

    lax.fori_loop(0, seq_per_worker, per_seq, 0)


def kernel(tokens, tok_emb, pos_emb):
    B, K, L = tokens.shape
    n_seq = B * K
    assert L == SEQ_LEN and n_seq % NUM_WORKERS == 0
    seq_per_worker = n_seq // NUM_WORKERS

    flat = tokens.reshape(n_seq * L).astype(jnp.int32)

    run = functools.partial(
        pl.kernel,
        out_type=jax.ShapeDtypeStruct((n_seq * L, D_MODEL), jnp.float32),
        mesh=plsc.VectorSubcoreMesh(core_axis_name="c", subcore_axis_name="s",
                                    num_cores=NUM_CORES,
                                    num_subcores=NUM_SUBCORES),
        scratch_types=[
            pltpu.VMEM((SEQ_LEN,), jnp.int32),
            pltpu.VMEM((SEQ_LEN,), jnp.int32),
            pltpu.VMEM((2, CHUNK, D_MODEL), jnp.float32),
            pltpu.VMEM((2, CHUNK, D_MODEL), jnp.float32),
            pltpu.VMEM((LANES,), jnp.int32),
            pltpu.SemaphoreType.DMA,
            pltpu.SemaphoreType.DMA,
            pltpu.SemaphoreType.DMA,
            pltpu.SemaphoreType.DMA,
            pltpu.SemaphoreType.DMA,
            pltpu.SemaphoreType.DMA,
        ],
        compiler_params=pltpu.CompilerParams(use_tc_tiling_on_sc=False,
                                             needs_layout_passes=False),
    )(functools.partial(_body, seq_per_worker=seq_per_worker))

    out = run(flat, tok_emb.astype(jnp.float32), pos_emb.astype(jnp.float32))
    return out.reshape(B, K, L, D_MODEL)


# speedup vs baseline: 1.0036x; 1.0036x over previous
import functools

import jax
import jax.numpy as jnp
from jax import lax
from jax.experimental import pallas as pl
from jax.experimental.pallas import tpu as pltpu, tpu_sc as plsc

D_MODEL = 64
SEQ_LEN = 1024
NUM_CORES = 2
NUM_SUBCORES = 16
NUM_WORKERS = NUM_CORES * NUM_SUBCORES
LANES = 16
CHUNK = 128
CHUNKS_PER_SEQ = SEQ_LEN // CHUNK


def _body(tok_hbm, te_hbm, pe_hbm, out_hbm,
          toks_v, pos_v, bt, bp, carry_v,
          sem_t0, sem_t1, sem_p0, sem_p1, sem_o0, sem_o1, seq_per_worker):
    wid = lax.axis_index("s") * NUM_CORES + lax.axis_index("c")
    sem_t = (sem_t0, sem_t1)
    sem_p = (sem_p0, sem_p1)
    sem_o = (sem_o0, sem_o1)

    def per_seq(i, _):
        s = wid * seq_per_worker + i
        base_tok = s * SEQ_LEN
        pltpu.sync_copy(tok_hbm.at[pl.ds(base_tok, SEQ_LEN)], toks_v)

        @plsc.parallel_loop(0, SEQ_LEN // LANES, step=1, unroll=4,
                            carry=jnp.zeros((LANES,), jnp.int32))
        def pos_grp(g, carry):
            t16 = toks_v[pl.ds(g * LANES, LANES)]
            m = jnp.minimum(t16, 1)
            cs = plsc.cumsum(m)
            pos_v[pl.ds(g * LANES, LANES)] = (cs + carry) * m
            return carry + lax.reduce_sum(m, axes=(0,))

        cp_t = [None, None]
        cp_p = [None, None]
        cp_o = [None, None]

        def issue(c):
            slot = c & 1
            base = c * CHUNK
            cp_t[slot] = pltpu.async_copy(
                te_hbm.at[toks_v.at[pl.ds(base, CHUNK)]],
                bt.at[slot], sem_t[slot])
            cp_p[slot] = pltpu.async_copy(
                pe_hbm.at[pos_v.at[pl.ds(base, CHUNK)]],
                bp.at[slot], sem_p[slot])

        issue(0)
        for c in range(CHUNKS_PER_SEQ):
            slot = c & 1
            other = 1 - slot
            if c + 1 < CHUNKS_PER_SEQ:
                if cp_o[other] is not None:
                    cp_o[other].wait()
                issue(c + 1)
            cp_t[slot].wait()
            cp_p[slot].wait()

            @plsc.parallel_loop(0, CHUNK, step=1, unroll=8)
            def add_row(r):
                for j in range(D_MODEL // LANES):
                    sl = pl.ds(j * LANES, LANES)
                    plsc.addupdate(bt.at[slot, r, sl], bp[slot, r, sl])

            cp_o[slot] = pltpu.async_copy(
                bt.at[slot], out_hbm.at[pl.ds(base_tok + c * CHUNK, CHUNK)],
                sem_o[slot])
        cp_o[0].wait()
        cp_o[1].wait()
        return 0

    lax.fori_loop(0, seq_per_worker, per_seq, 0)


def kernel(tokens, tok_emb, pos_emb):
    B, K, L = tokens.shape
    n_seq = B * K
    assert L == SEQ_LEN and n_seq % NUM_WORKERS == 0
    seq_per_worker = n_seq // NUM_WORKERS

    flat = tokens.reshape(n_seq * L).astype(jnp.int32)

    run = functools.partial(
        pl.kernel,
        out_type=jax.ShapeDtypeStruct((n_seq * L, D_MODEL), jnp.float32),
        mesh=plsc.VectorSubcoreMesh(core_axis_name="c", subcore_axis_name="s",
                                    num_cores=NUM_CORES,
                                    num_subcores=NUM_SUBCORES),
        scratch_types=[
            pltpu.VMEM((SEQ_LEN,), jnp.int32),
            pltpu.VMEM((SEQ_LEN,), jnp.int32),
            pltpu.VMEM((2, CHUNK, D_MODEL), jnp.float32),
            pltpu.VMEM((2, CHUNK, D_MODEL), jnp.float32),
            pltpu.VMEM((LANES,), jnp.int32),
            pltpu.SemaphoreType.DMA,
            pltpu.SemaphoreType.DMA,
            pltpu.SemaphoreType.DMA,
            pltpu.SemaphoreType.DMA,
            pltpu.SemaphoreType.DMA,
            pltpu.SemaphoreType.DMA,
        ],
        compiler_params=pltpu.CompilerParams(use_tc_tiling_on_sc=False,
                                             needs_layout_passes=False),
    )(functools.partial(_body, seq_per_worker=seq_per_worker))

    out = run(flat, tok_emb.astype(jnp.float32), pos_emb.astype(jnp.float32))
    return out.reshape(B, K, L, D_MODEL)
